# baseline (speedup 1.0000x reference)
import functools

import jax
import jax.numpy as jnp
from jax import lax
from jax.experimental import pallas as pl
from jax.experimental.pallas import tpu as pltpu
from jax.experimental.pallas import tpu_sc as plsc

N_E = 1024
E_DIM = 64
Z_CH = 192
BETA = 0.25
NB = 2
T = 16384
TT = NB * T

BT1 = 8192
BT2 = 4096

NW = 32
CH = 128
NCH = TT // (NW * CH)
BPW = NCH * CH
PD = 128
NPASS = 4
NCHP = NCH // NPASS
HBPW = BPW // NPASS


def _e2_tree(sq):
    acc = sq[:, 0:8]
    for k in range(1, 8):
        acc = acc + sq[:, 8 * k:8 * k + 8]
    h = acc[:, 0:4] + acc[:, 4:8]
    h = h[:, 0:2] + h[:, 2:4]
    return h[:, 0:1] + h[:, 1:2]


def _z2_tree(sq):
    chunks = []
    for c in range(8):
        r = [sq[8 * c + i:8 * c + i + 1, :] for i in range(8)]
        l1 = [r[0] + r[1], r[2] + r[3], r[4] + r[5], r[6] + r[7]]
        l2 = [l1[0] + l1[1], l1[2] + l1[3]]
        chunks.append(l2[0] + l2[1])
    acc = chunks[0]
    for c in range(1, 8):
        acc = acc + chunks[c]
    return acc


def _stage1_body(z_ref, wq_ref, bq_ref, emb_ref,
                 idx_ref, loss_ref, emb_pad_ref, e2_ref):
    b = pl.program_id(0)
    t = pl.program_id(1)

    @pl.when((b == 0) & (t == 0))
    def _():
        embv = emb_ref[...]
        e2_ref[...] = _e2_tree(embv * embv)
        emb_pad_ref[...] = jnp.concatenate(
            [embv, jnp.zeros((N_E, PD - E_DIM), jnp.float32)], axis=1)
        loss_ref[...] = jnp.zeros((1, 1), jnp.float32)

    zp = jnp.dot(wq_ref[...], z_ref[0],
                 preferred_element_type=jnp.float32) + bq_ref[...]
    emb = emb_ref[...]
    e2 = e2_ref[...]
    z2 = _z2_tree(zp * zp)
    s = jnp.dot(emb, zp, preferred_element_type=jnp.float32)
    d = (z2 + e2) - 2.0 * s
    m = jnp.min(d, axis=0, keepdims=True)
    rows = lax.broadcasted_iota(jnp.int32, d.shape, 0)
    idx = jnp.min(jnp.where(d == m, rows, N_E), axis=0,
                  keepdims=True)
    for i in range(BT1 // CH):
        idx_ref[i:i + 1, :] = idx[:, CH * i:CH * (i + 1)]
    loss_ref[...] += jnp.sum(m).reshape(1, 1)


def _stage3_body(zq_ref, wp_ref, bp_ref, out_ref):
    zq = zq_ref[0][:, :E_DIM]
    out_ref[0] = lax.dot_general(
        wp_ref[...], zq, (((1,), (1,)), ((), ())),
        preferred_element_type=jnp.float32) + bp_ref[...]


@functools.cache
def _make_sc_gather():
    @functools.partial(
        pl.kernel,
        mesh=plsc.VectorSubcoreMesh(core_axis_name="c", subcore_axis_name="s"),
        out_type=jax.ShapeDtypeStruct((TT, PD), jnp.float32),
        scratch_types=[
            pltpu.VMEM((NCH, CH), jnp.int32),
            pltpu.VMEM((2, HBPW, PD), jnp.float32),
            pltpu.SemaphoreType.DMA,
            pltpu.SemaphoreType.DMA,
        ],
    )
    def _sc_gather(emb_hbm, idx_hbm, out_hbm, idx_v, rows_v, gsem, osem):
        wid = lax.axis_index("s") * 2 + lax.axis_index("c")
        base = wid * BPW
        pltpu.sync_copy(idx_hbm.at[pl.ds(wid * NCH, NCH)], idx_v)
        outs = []
        for p in range(NPASS):
            if p >= 2:
                outs[p - 2].wait()
            gathers = []
            for j in range(NCHP):
                gathers.append(pltpu.async_copy(
                    emb_hbm.at[idx_v.at[p * NCHP + j]],
                    rows_v.at[p % 2, pl.ds(j * CH, CH)], gsem))
            for g in gathers:
                g.wait()
            outs.append(pltpu.async_copy(
                rows_v.at[p % 2], out_hbm.at[pl.ds(base + p * HBPW, HBPW)],
                osem))
        outs[NPASS - 2].wait()
        outs[NPASS - 1].wait()

    return _sc_gather


def _stage1(z3, Wq, bq2, embedding, interpret=False):
    return pl.pallas_call(
        _stage1_body,
        grid=(NB, T // BT1),
        in_specs=[
            pl.BlockSpec((1, Z_CH, BT1), lambda b, t: (b, 0, t)),
            pl.BlockSpec((E_DIM, Z_CH), lambda b, t: (0, 0)),
            pl.BlockSpec((E_DIM, 1), lambda b, t: (0, 0)),
            pl.BlockSpec((N_E, E_DIM), lambda b, t: (0, 0)),
        ],
        out_specs=[
            pl.BlockSpec((BT1 // CH, CH), lambda b, t: (b * (T // BT1) + t, 0)),
            pl.BlockSpec((1, 1), lambda b, t: (0, 0)),
            pl.BlockSpec((N_E, PD), lambda b, t: (0, 0)),
        ],
        out_shape=[
            jax.ShapeDtypeStruct((TT // CH, CH), jnp.int32),
            jax.ShapeDtypeStruct((1, 1), jnp.float32),
            jax.ShapeDtypeStruct((N_E, PD), jnp.float32),
        ],
        scratch_shapes=[pltpu.VMEM((N_E, 1), jnp.float32)],
        compiler_params=pltpu.CompilerParams(
            dimension_semantics=("arbitrary", "arbitrary")),
        interpret=interpret,
    )(z3, Wq, bq2, embedding)


def _stage3(zq3, Wp, bp2, interpret=False):
    return pl.pallas_call(
        _stage3_body,
        grid=(NB, T // BT2),
        in_specs=[
            pl.BlockSpec((1, BT2, PD), lambda b, t: (b, t, 0)),
            pl.BlockSpec((Z_CH, E_DIM), lambda b, t: (0, 0)),
            pl.BlockSpec((Z_CH, 1), lambda b, t: (0, 0)),
        ],
        out_specs=pl.BlockSpec((1, Z_CH, BT2), lambda b, t: (b, 0, t)),
        out_shape=jax.ShapeDtypeStruct((NB, Z_CH, T), jnp.float32),
        compiler_params=pltpu.CompilerParams(
            dimension_semantics=("arbitrary", "arbitrary")),
        interpret=interpret,
    )(zq3, Wp, bp2)


def kernel(z, embedding, Wq, bq, Wp, bp):
    z3 = z.reshape(NB, Z_CH, T)
    idx2, loss_acc, emb_pad = _stage1(z3, Wq, bq.reshape(E_DIM, 1), embedding)

    zq = _make_sc_gather()(emb_pad, idx2)

    out3 = _stage3(zq.reshape(NB, T, PD), Wp, bp.reshape(Z_CH, 1))

    mean_d = loss_acc[0, 0] / (TT * E_DIM)
    loss = mean_d + BETA * mean_d
    return out3.reshape(z.shape), loss, idx2.reshape(TT)


# speedup vs baseline: 1.0111x; 1.0111x over previous
import functools

import jax
import jax.numpy as jnp
from jax import lax
from jax.experimental import pallas as pl
from jax.experimental.pallas import tpu as pltpu
from jax.experimental.pallas import tpu_sc as plsc

N_E = 1024
E_DIM = 64
Z_CH = 192
BETA = 0.25
NB = 2
T = 16384
TT = NB * T

BT1 = 8192
BT2 = 4096

NW = 32
CH = 128
NCH = TT // (NW * CH)
BPW = NCH * CH
PD = 128
NPASS = 2
NCHP = NCH // NPASS
HBPW = BPW // NPASS


def _e2_tree(sq):
    acc = sq[:, 0:8]
    for k in range(1, 8):
        acc = acc + sq[:, 8 * k:8 * k + 8]
    h = acc[:, 0:4] + acc[:, 4:8]
    h = h[:, 0:2] + h[:, 2:4]
    return h[:, 0:1] + h[:, 1:2]


def _z2_tree(sq):
    chunks = []
    for c in range(8):
        r = [sq[8 * c + i:8 * c + i + 1, :] for i in range(8)]
        l1 = [r[0] + r[1], r[2] + r[3], r[4] + r[5], r[6] + r[7]]
        l2 = [l1[0] + l1[1], l1[2] + l1[3]]
        chunks.append(l2[0] + l2[1])
    acc = chunks[0]
    for c in range(1, 8):
        acc = acc + chunks[c]
    return acc


def _stage1_body(z_ref, wq_ref, bq_ref, emb_ref,
                 idx_ref, loss_ref, emb_pad_ref, e2_ref):
    b = pl.program_id(0)
    t = pl.program_id(1)

    @pl.when((b == 0) & (t == 0))
    def _():
        embv = emb_ref[...]
        e2_ref[...] = _e2_tree(embv * embv)
        emb_pad_ref[...] = jnp.concatenate(
            [embv, jnp.zeros((N_E, PD - E_DIM), jnp.float32)], axis=1)
        loss_ref[...] = jnp.zeros((1, 1), jnp.float32)

    zp = jnp.dot(wq_ref[...], z_ref[0],
                 preferred_element_type=jnp.float32) + bq_ref[...]
    emb = emb_ref[...]
    e2 = e2_ref[...]
    z2 = _z2_tree(zp * zp)
    s = jnp.dot(emb, zp, preferred_element_type=jnp.float32)
    d = (z2 + e2) - 2.0 * s
    m = jnp.min(d, axis=0, keepdims=True)
    rows = lax.broadcasted_iota(jnp.int32, d.shape, 0)
    idx = jnp.min(jnp.where(d == m, rows, N_E), axis=0,
                  keepdims=True)
    for i in range(BT1 // CH):
        idx_ref[i:i + 1, :] = idx[:, CH * i:CH * (i + 1)]
    loss_ref[...] += jnp.sum(m).reshape(1, 1)


def _stage3_body(zq_ref, wp_ref, bp_ref, out_ref):
    zq = zq_ref[0][:, :E_DIM]
    out_ref[0] = lax.dot_general(
        wp_ref[...], zq, (((1,), (1,)), ((), ())),
        preferred_element_type=jnp.float32) + bp_ref[...]


@functools.cache
def _make_sc_gather():
    @functools.partial(
        pl.kernel,
        mesh=plsc.VectorSubcoreMesh(core_axis_name="c", subcore_axis_name="s"),
        out_type=jax.ShapeDtypeStruct((TT, PD), jnp.float32),
        scratch_types=[
            pltpu.VMEM((NCH, CH), jnp.int32),
            pltpu.VMEM((HBPW, PD), jnp.float32),
            pltpu.SemaphoreType.DMA,
        ],
    )
    def _sc_gather(emb_hbm, idx_hbm, out_hbm, idx_v, rows_v, sem):
        wid = lax.axis_index("s") * 2 + lax.axis_index("c")
        base = wid * BPW
        pltpu.sync_copy(idx_hbm.at[pl.ds(wid * NCH, NCH)], idx_v)
        for p in range(NPASS):
            copies = []
            for j in range(NCHP):
                copies.append(pltpu.async_copy(
                    emb_hbm.at[idx_v.at[p * NCHP + j]],
                    rows_v.at[pl.ds(j * CH, CH)], sem))
            for c in copies:
                c.wait()
            pltpu.sync_copy(rows_v, out_hbm.at[pl.ds(base + p * HBPW, HBPW)])

    return _sc_gather


def _stage1(z3, Wq, bq2, embedding, interpret=False):
    return pl.pallas_call(
        _stage1_body,
        grid=(NB, T // BT1),
        in_specs=[
            pl.BlockSpec((1, Z_CH, BT1), lambda b, t: (b, 0, t)),
            pl.BlockSpec((E_DIM, Z_CH), lambda b, t: (0, 0)),
            pl.BlockSpec((E_DIM, 1), lambda b, t: (0, 0)),
            pl.BlockSpec((N_E, E_DIM), lambda b, t: (0, 0)),
        ],
        out_specs=[
            pl.BlockSpec((BT1 // CH, CH), lambda b, t: (b * (T // BT1) + t, 0)),
            pl.BlockSpec((1, 1), lambda b, t: (0, 0)),
            pl.BlockSpec((N_E, PD), lambda b, t: (0, 0)),
        ],
        out_shape=[
            jax.ShapeDtypeStruct((TT // CH, CH), jnp.int32),
            jax.ShapeDtypeStruct((1, 1), jnp.float32),
            jax.ShapeDtypeStruct((N_E, PD), jnp.float32),
        ],
        scratch_shapes=[pltpu.VMEM((N_E, 1), jnp.float32)],
        compiler_params=pltpu.CompilerParams(
            dimension_semantics=("arbitrary", "arbitrary")),
        interpret=interpret,
    )(z3, Wq, bq2, embedding)


def _stage3(zq3, Wp, bp2, interpret=False):
    return pl.pallas_call(
        _stage3_body,
        grid=(NB, T // BT2),
        in_specs=[
            pl.BlockSpec((1, BT2, PD), lambda b, t: (b, t, 0)),
            pl.BlockSpec((Z_CH, E_DIM), lambda b, t: (0, 0)),
            pl.BlockSpec((Z_CH, 1), lambda b, t: (0, 0)),
        ],
        out_specs=pl.BlockSpec((1, Z_CH, BT2), lambda b, t: (b, 0, t)),
        out_shape=jax.ShapeDtypeStruct((NB, Z_CH, T), jnp.float32),
        compiler_params=pltpu.CompilerParams(
            dimension_semantics=("arbitrary", "arbitrary")),
        interpret=interpret,
    )(zq3, Wp, bp2)


def kernel(z, embedding, Wq, bq, Wp, bp):
    z3 = z.reshape(NB, Z_CH, T)
    idx2, loss_acc, emb_pad = _stage1(z3, Wq, bq.reshape(E_DIM, 1), embedding)

    zq = _make_sc_gather()(emb_pad, idx2)

    out3 = _stage3(zq.reshape(NB, T, PD), Wp, bp.reshape(Z_CH, 1))

    mean_d = loss_acc[0, 0] / (TT * E_DIM)
    loss = mean_d + BETA * mean_d
    return out3.reshape(z.shape), loss, idx2.reshape(TT)
